# baseline (speedup 1.0000x reference)
import jax
import jax.numpy as jnp
from jax.experimental import pallas as pl
from jax.experimental.pallas import tpu as pltpu

NUM_CLASSES = 13
BN_EPS = 1e-5


def _square_distance(src, dst):
    return (jnp.sum(src ** 2, -1)[:, :, None] + jnp.sum(dst ** 2, -1)[:, None, :]
            - 2.0 * jnp.einsum('bnc,bmc->bnm', src, dst))


def _index_points(points, idx):
    return jax.vmap(lambda p, i: p[i])(points, idx)


def _farthest_point_sample(xyz, npoint):
    xyz = jax.lax.stop_gradient(xyz)
    b, n, _ = xyz.shape

    def step(carry, _):
        distance, farthest = carry
        centroid = jnp.take_along_axis(xyz, farthest[:, None, None], axis=1)
        dist = jnp.sum((xyz - centroid) ** 2, -1)
        distance = jnp.minimum(distance, dist)
        new_far = jnp.argmax(distance, -1).astype(jnp.int32)
        return (distance, new_far), farthest

    init = (jnp.full((b, n), 1e10, dtype=xyz.dtype), jnp.zeros((b,), jnp.int32))
    _, cents = jax.lax.scan(step, init, None, length=npoint)
    return jnp.transpose(cents, (1, 0))


def _query_ball_point(radius, nsample, xyz, new_xyz):
    b, n, _ = xyz.shape
    s = new_xyz.shape[1]
    sqrdists = _square_distance(new_xyz, xyz)
    group_idx = jnp.broadcast_to(jnp.arange(n, dtype=jnp.int32), (b, s, n))
    group_idx = jnp.where(sqrdists > radius ** 2, n, group_idx)
    group_idx = jnp.sort(group_idx, axis=-1)[:, :, :nsample]
    group_first = jnp.broadcast_to(group_idx[:, :, :1], group_idx.shape)
    group_idx = jnp.where(group_idx == n, group_first, group_idx)
    return group_idx


def _bn_relu(x, p):
    x = x @ p['W'].T + p['b']
    x = x / jnp.sqrt(1.0 + BN_EPS) * p['g'] + p['be']
    return jax.nn.relu(x)


def _set_abstraction(xyz, points, npoint, radius, nsample, layers):
    xyz_t = jnp.transpose(xyz, (0, 2, 1))
    fps_idx = _farthest_point_sample(xyz_t, npoint)
    new_xyz = _index_points(xyz_t, fps_idx)
    idx = _query_ball_point(radius, nsample, xyz_t, new_xyz)
    grouped_xyz = _index_points(xyz_t, idx) - new_xyz[:, :, None, :]
    if points is not None:
        pts_t = jnp.transpose(points, (0, 2, 1))
        grouped = jnp.concatenate([grouped_xyz, _index_points(pts_t, idx)], axis=-1)
    else:
        grouped = grouped_xyz
    x = grouped
    for p in layers:
        x = _bn_relu(x, p)
    x = jnp.max(x, axis=2)
    return jnp.transpose(new_xyz, (0, 2, 1)), jnp.transpose(x, (0, 2, 1))


def _feature_propagation(xyz1, xyz2, points1, points2, layers):
    x1 = jnp.transpose(xyz1, (0, 2, 1))
    x2 = jnp.transpose(xyz2, (0, 2, 1))
    p2 = jnp.transpose(points2, (0, 2, 1))
    b, n, _ = x1.shape
    s = x2.shape[1]
    if s == 1:
        interpolated = jnp.repeat(p2, n, axis=1)
    else:
        dists = _square_distance(x1, x2)
        idx = jnp.argsort(dists, axis=-1)[:, :, :3]
        d = jnp.take_along_axis(dists, idx, axis=-1)
        dist_recip = 1.0 / (d + 1e-8)
        norm = jnp.sum(dist_recip, axis=2, keepdims=True)
        weight = dist_recip / norm
        interpolated = jnp.sum(_index_points(p2, idx) * weight[..., None], axis=2)
    if points1 is not None:
        p1 = jnp.transpose(points1, (0, 2, 1))
        new_points = jnp.concatenate([p1, interpolated], axis=-1)
    else:
        new_points = interpolated
    x = new_points
    for p in layers:
        x = _bn_relu(x, p)
    return jnp.transpose(x, (0, 2, 1))


def _head_body(x_ref, w1_ref, b1_ref, g1_ref, be1_ref, w2_ref, b2_ref, out_ref):
    x = x_ref[0]
    h = jax.lax.dot_general(x, w1_ref[...], (((1,), (1,)), ((), ())),
                            preferred_element_type=jnp.float32) + b1_ref[...]
    h = h / jnp.sqrt(1.0 + BN_EPS) * g1_ref[...] + be1_ref[...]
    h = jax.nn.relu(h)
    logits = jax.lax.dot_general(h, w2_ref[...], (((1,), (1,)), ((), ())),
                                 preferred_element_type=jnp.float32) + b2_ref[...]
    out_ref[0] = jax.nn.log_softmax(logits, axis=-1)


def _head(t, p1, p2):
    b, n, c = t.shape
    blk = 1024
    grid = (b, n // blk)
    return pl.pallas_call(
        _head_body,
        grid=grid,
        in_specs=[
            pl.BlockSpec((1, blk, c), lambda i, j: (i, j, 0)),
            pl.BlockSpec((128, c), lambda i, j: (0, 0)),
            pl.BlockSpec((128,), lambda i, j: (0,)),
            pl.BlockSpec((128,), lambda i, j: (0,)),
            pl.BlockSpec((128,), lambda i, j: (0,)),
            pl.BlockSpec((NUM_CLASSES, c), lambda i, j: (0, 0)),
            pl.BlockSpec((NUM_CLASSES,), lambda i, j: (0,)),
        ],
        out_specs=pl.BlockSpec((1, blk, NUM_CLASSES), lambda i, j: (i, j, 0)),
        out_shape=jax.ShapeDtypeStruct((b, n, NUM_CLASSES), jnp.float32),
    )(t, p1['W'], p1['b'], p1['g'], p1['be'], p2['W'], p2['b'])


def kernel(xyz, params):
    l0_points = xyz
    l0_xyz = xyz[:, :3, :]
    l1_xyz, l1_points = _set_abstraction(l0_xyz, l0_points, 1024, 0.1, 32, params['sa1'])
    l2_xyz, l2_points = _set_abstraction(l1_xyz, l1_points, 256, 0.2, 32, params['sa2'])
    l3_xyz, l3_points = _set_abstraction(l2_xyz, l2_points, 64, 0.4, 32, params['sa3'])
    l4_xyz, l4_points = _set_abstraction(l3_xyz, l3_points, 16, 0.8, 32, params['sa4'])
    l3_points = _feature_propagation(l3_xyz, l4_xyz, l3_points, l4_points, params['fp4'])
    l2_points = _feature_propagation(l2_xyz, l3_xyz, l2_points, l3_points, params['fp3'])
    l1_points = _feature_propagation(l1_xyz, l2_xyz, l1_points, l2_points, params['fp2'])
    l0_out = _feature_propagation(l0_xyz, l1_xyz, None, l1_points, params['fp1'])
    t = jnp.transpose(l0_out, (0, 2, 1))
    return _head(t, params['head1'], params['head2'])


# speedup vs baseline: 3.7654x; 3.7654x over previous
import functools

import jax
import jax.numpy as jnp
import numpy as np
from jax.experimental import pallas as pl
from jax.experimental.pallas import tpu as pltpu

NUM_CLASSES = 13
BN_EPS = 1e-5
_BNS = float(1.0 / np.sqrt(np.float32(1.0 + BN_EPS), dtype=np.float32))


def _fps_body(x_ref, y_ref, z_ref, out_ref, dist_ref):
    b, n = x_ref.shape
    x = x_ref[...]
    y = y_ref[...]
    z = z_ref[...]
    iota = jax.lax.broadcasted_iota(jnp.int32, (b, n), 1)
    dist_ref[...] = jnp.full((b, n), 1e10, jnp.float32)
    npoint = out_ref.shape[0]

    def step(t, far):
        oh = iota == far
        cx = jnp.sum(jnp.where(oh, x, 0.0), axis=1, keepdims=True)
        cy = jnp.sum(jnp.where(oh, y, 0.0), axis=1, keepdims=True)
        cz = jnp.sum(jnp.where(oh, z, 0.0), axis=1, keepdims=True)
        out_ref[t] = jnp.concatenate([cx, cy, cz], axis=1)
        dx = x - cx
        dy = y - cy
        dz = z - cz
        d = dx * dx + dy * dy + dz * dz
        dist = jnp.minimum(dist_ref[...], d)
        dist_ref[...] = dist
        m = jnp.max(dist, axis=1, keepdims=True)
        return jnp.min(jnp.where(dist == m, iota, n), axis=1, keepdims=True)

    jax.lax.fori_loop(0, npoint, step, jnp.zeros((b, 1), jnp.int32))


def _fps(xyz_c, npoint):
    b, _, n = xyz_c.shape
    out = pl.pallas_call(
        _fps_body,
        grid=(1,),
        in_specs=[pl.BlockSpec((b, n), lambda i: (0, 0))] * 3,
        out_specs=pl.BlockSpec((npoint, b, 3), lambda i: (0, 0, 0)),
        out_shape=jax.ShapeDtypeStruct((npoint, b, 3), jnp.float32),
        scratch_shapes=[pltpu.VMEM((b, n), jnp.float32)],
    )(xyz_c[:, 0, :], xyz_c[:, 1, :], xyz_c[:, 2, :])
    return jnp.transpose(out, (1, 0, 2))


def _bq_body(nx_ref, xyzc_ref, out_ref, *, radius, nsample):
    nx = nx_ref[0]
    xyzc = xyzc_ref[0]
    sb = nx.shape[0]
    n = xyzc.shape[1]
    src2 = jnp.sum(nx * nx, axis=1, keepdims=True)
    dst2 = jnp.sum(xyzc * xyzc, axis=0, keepdims=True)
    cross = jax.lax.dot_general(nx, xyzc, (((1,), (0,)), ((), ())),
                                preferred_element_type=jnp.float32)
    sq = src2 + dst2 - 2.0 * cross
    iota = jax.lax.broadcasted_iota(jnp.int32, (sb, n), 1)
    keys = jnp.where(sq > radius * radius, n, iota)
    cols = []
    for _ in range(nsample):
        m = jnp.min(keys, axis=1, keepdims=True)
        cols.append(m)
        keys = jnp.where(keys == m, n, keys)
    idx = jnp.concatenate(cols, axis=1)
    idx = jnp.where(idx == n, cols[0], idx)
    out_ref[0] = idx


def _bq(new_xyz, xyz_c, radius, nsample, sb):
    b, s, _ = new_xyz.shape
    n = xyz_c.shape[2]
    grid = (b, s // sb)
    return pl.pallas_call(
        functools.partial(_bq_body, radius=radius, nsample=nsample),
        grid=grid,
        in_specs=[
            pl.BlockSpec((1, sb, 3), lambda i, j: (i, j, 0)),
            pl.BlockSpec((1, 3, n), lambda i, j: (i, 0, 0)),
        ],
        out_specs=pl.BlockSpec((1, sb, nsample), lambda i, j: (i, j, 0)),
        out_shape=jax.ShapeDtypeStruct((b, s, nsample), jnp.int32),
    )(new_xyz, xyz_c)


def _sa_mlp_body(g_ref, *refs):
    nl = (len(refs) - 1) // 4
    out_ref = refs[-1]
    _, sb, ns, c = g_ref.shape
    x = g_ref[0].reshape(sb * ns, c)
    for li in range(nl):
        w, bb, g, be = refs[4 * li:4 * li + 4]
        x = jax.lax.dot_general(x, w[...], (((1,), (1,)), ((), ())),
                                preferred_element_type=jnp.float32) + bb[...]
        x = x * _BNS * g[...] + be[...]
        x = jnp.maximum(x, 0.0)
    x = x.reshape(sb, ns, x.shape[-1])
    out_ref[0] = jnp.max(x, axis=1)


def _sa_mlp(grouped, layers, sb):
    b, s, ns, c = grouped.shape
    cout = layers[-1]['W'].shape[0]
    grid = (b, s // sb)
    in_specs = [pl.BlockSpec((1, sb, ns, c), lambda i, j: (i, j, 0, 0))]
    args = [grouped]
    for p in layers:
        co, ci = p['W'].shape
        in_specs += [
            pl.BlockSpec((co, ci), lambda i, j: (0, 0)),
            pl.BlockSpec((co,), lambda i, j: (0,)),
            pl.BlockSpec((co,), lambda i, j: (0,)),
            pl.BlockSpec((co,), lambda i, j: (0,)),
        ]
        args += [p['W'], p['b'], p['g'], p['be']]
    return pl.pallas_call(
        _sa_mlp_body,
        grid=grid,
        in_specs=in_specs,
        out_specs=pl.BlockSpec((1, sb, cout), lambda i, j: (i, j, 0)),
        out_shape=jax.ShapeDtypeStruct((b, s, cout), jnp.float32),
    )(*args)


def _fp_body(x1_ref, x2c_ref, p2_ref, *refs, has_p1, nl):
    if has_p1:
        p1_ref = refs[0]
        refs = refs[1:]
    out_ref = refs[-1]
    x1 = x1_ref[0]
    x2c = x2c_ref[0]
    p2 = p2_ref[0]
    nb = x1.shape[0]
    s = x2c.shape[1]
    src2 = jnp.sum(x1 * x1, axis=1, keepdims=True)
    dst2 = jnp.sum(x2c * x2c, axis=0, keepdims=True)
    cross = jax.lax.dot_general(x1, x2c, (((1,), (0,)), ((), ())),
                                preferred_element_type=jnp.float32)
    d = src2 + dst2 - 2.0 * cross
    iota = jax.lax.broadcasted_iota(jnp.int32, (nb, s), 1)
    ws = []
    poss = []
    for _ in range(3):
        m = jnp.min(d, axis=1, keepdims=True)
        pos = jnp.min(jnp.where(d == m, iota, s), axis=1, keepdims=True)
        ws.append(1.0 / (m + 1e-8))
        poss.append(pos)
        d = jnp.where(iota == pos, jnp.float32(3.4e38), d)
    norm = ws[0] + ws[1] + ws[2]
    wmat = jnp.zeros((nb, s), jnp.float32)
    for k in range(3):
        wmat = jnp.where(iota == poss[k], ws[k] / norm, wmat)
    x = jax.lax.dot_general(wmat, p2, (((1,), (0,)), ((), ())),
                            preferred_element_type=jnp.float32)
    if has_p1:
        x = jnp.concatenate([p1_ref[0], x], axis=1)
    for li in range(nl):
        w, bb, g, be = refs[4 * li:4 * li + 4]
        x = jax.lax.dot_general(x, w[...], (((1,), (1,)), ((), ())),
                                preferred_element_type=jnp.float32) + bb[...]
        x = x * _BNS * g[...] + be[...]
        x = jnp.maximum(x, 0.0)
    out_ref[0] = x


def _fp(x1, x2_c, p1, p2, layers, nb):
    b, n, _ = x1.shape
    s = x2_c.shape[2]
    c2 = p2.shape[2]
    cout = layers[-1]['W'].shape[0]
    grid = (b, n // nb)
    in_specs = [
        pl.BlockSpec((1, nb, 3), lambda i, j: (i, j, 0)),
        pl.BlockSpec((1, 3, s), lambda i, j: (i, 0, 0)),
        pl.BlockSpec((1, s, c2), lambda i, j: (i, 0, 0)),
    ]
    args = [x1, x2_c, p2]
    if p1 is not None:
        in_specs.append(pl.BlockSpec((1, nb, p1.shape[2]), lambda i, j: (i, j, 0)))
        args.append(p1)
    for p in layers:
        co, ci = p['W'].shape
        in_specs += [
            pl.BlockSpec((co, ci), lambda i, j: (0, 0)),
            pl.BlockSpec((co,), lambda i, j: (0,)),
            pl.BlockSpec((co,), lambda i, j: (0,)),
            pl.BlockSpec((co,), lambda i, j: (0,)),
        ]
        args += [p['W'], p['b'], p['g'], p['be']]
    return pl.pallas_call(
        functools.partial(_fp_body, has_p1=p1 is not None, nl=len(layers)),
        grid=grid,
        in_specs=in_specs,
        out_specs=pl.BlockSpec((1, nb, cout), lambda i, j: (i, j, 0)),
        out_shape=jax.ShapeDtypeStruct((b, n, cout), jnp.float32),
    )(*args)


def _head_body(x_ref, w1_ref, b1_ref, g1_ref, be1_ref, w2_ref, b2_ref, out_ref):
    x = x_ref[0]
    h = jax.lax.dot_general(x, w1_ref[...], (((1,), (1,)), ((), ())),
                            preferred_element_type=jnp.float32) + b1_ref[...]
    h = h * _BNS * g1_ref[...] + be1_ref[...]
    h = jnp.maximum(h, 0.0)
    logits = jax.lax.dot_general(h, w2_ref[...], (((1,), (1,)), ((), ())),
                                 preferred_element_type=jnp.float32) + b2_ref[...]
    out_ref[0] = jax.nn.log_softmax(logits, axis=-1)


def _head(t, p1, p2):
    b, n, c = t.shape
    blk = 1024
    grid = (b, n // blk)
    return pl.pallas_call(
        _head_body,
        grid=grid,
        in_specs=[
            pl.BlockSpec((1, blk, c), lambda i, j: (i, j, 0)),
            pl.BlockSpec((128, c), lambda i, j: (0, 0)),
            pl.BlockSpec((128,), lambda i, j: (0,)),
            pl.BlockSpec((128,), lambda i, j: (0,)),
            pl.BlockSpec((128,), lambda i, j: (0,)),
            pl.BlockSpec((NUM_CLASSES, c), lambda i, j: (0, 0)),
            pl.BlockSpec((NUM_CLASSES,), lambda i, j: (0,)),
        ],
        out_specs=pl.BlockSpec((1, blk, NUM_CLASSES), lambda i, j: (i, j, 0)),
        out_shape=jax.ShapeDtypeStruct((b, n, NUM_CLASSES), jnp.float32),
    )(t, p1['W'], p1['b'], p1['g'], p1['be'], p2['W'], p2['b'])


def _index_points(points, idx):
    return jax.vmap(lambda p, i: p[i])(points, idx)


def _sa(xyz_c, pts_t, npoint, radius, nsample, layers, sb):
    new_xyz = _fps(xyz_c, npoint)
    idx = _bq(new_xyz, xyz_c, radius, nsample, sb)
    xyz_t = jnp.transpose(xyz_c, (0, 2, 1))
    table = jnp.concatenate([xyz_t, pts_t], axis=-1)
    g = _index_points(table, idx)
    grouped = jnp.concatenate(
        [g[..., :3] - new_xyz[:, :, None, :], g[..., 3:]], axis=-1)
    return new_xyz, _sa_mlp(grouped, layers, sb)


def kernel(xyz, params):
    l0_xyz_c = xyz[:, :3, :]
    l0_pts_t = jnp.transpose(xyz, (0, 2, 1))

    l1_xyz, l1_pts = _sa(l0_xyz_c, l0_pts_t, 1024, 0.1, 32, params['sa1'], sb=256)
    l1_xyz_c = jnp.transpose(l1_xyz, (0, 2, 1))
    l2_xyz, l2_pts = _sa(l1_xyz_c, l1_pts, 256, 0.2, 32, params['sa2'], sb=256)
    l2_xyz_c = jnp.transpose(l2_xyz, (0, 2, 1))
    l3_xyz, l3_pts = _sa(l2_xyz_c, l2_pts, 64, 0.4, 32, params['sa3'], sb=64)
    l3_xyz_c = jnp.transpose(l3_xyz, (0, 2, 1))
    l4_xyz, l4_pts = _sa(l3_xyz_c, l3_pts, 16, 0.8, 32, params['sa4'], sb=16)
    l4_xyz_c = jnp.transpose(l4_xyz, (0, 2, 1))

    l3_pts = _fp(l3_xyz, l4_xyz_c, l3_pts, l4_pts, params['fp4'], nb=64)
    l2_pts = _fp(l2_xyz, l3_xyz_c, l2_pts, l3_pts, params['fp3'], nb=256)
    l1_pts = _fp(l1_xyz, l2_xyz_c, l1_pts, l2_pts, params['fp2'], nb=512)
    l0_out = _fp(l0_pts_t[..., :3], l1_xyz_c, None, l1_pts, params['fp1'], nb=512)

    return _head(l0_out, params['head1'], params['head2'])


# speedup vs baseline: 16.3853x; 4.3515x over previous
import functools

import jax
import jax.numpy as jnp
import numpy as np
from jax import lax
from jax.experimental import pallas as pl
from jax.experimental.pallas import tpu as pltpu
from jax.experimental.pallas import tpu_sc as plsc

NUM_CLASSES = 13
BN_EPS = 1e-5
_BNS = float(1.0 / np.sqrt(np.float32(1.0 + BN_EPS), dtype=np.float32))


def _fps_body(x_ref, y_ref, z_ref, out_ref, dist_ref):
    b, n = x_ref.shape
    x = x_ref[...]
    y = y_ref[...]
    z = z_ref[...]
    iota = jax.lax.broadcasted_iota(jnp.int32, (b, n), 1)
    dist_ref[...] = jnp.full((b, n), 1e10, jnp.float32)
    npoint = out_ref.shape[0]

    def step(t, far):
        oh = iota == far
        cx = jnp.sum(jnp.where(oh, x, 0.0), axis=1, keepdims=True)
        cy = jnp.sum(jnp.where(oh, y, 0.0), axis=1, keepdims=True)
        cz = jnp.sum(jnp.where(oh, z, 0.0), axis=1, keepdims=True)
        out_ref[t] = jnp.concatenate([cx, cy, cz], axis=1)
        dx = x - cx
        dy = y - cy
        dz = z - cz
        d = dx * dx + dy * dy + dz * dz
        dist = jnp.minimum(dist_ref[...], d)
        dist_ref[...] = dist
        m = jnp.max(dist, axis=1, keepdims=True)
        return jnp.min(jnp.where(dist == m, iota, n), axis=1, keepdims=True)

    jax.lax.fori_loop(0, npoint, step, jnp.zeros((b, 1), jnp.int32))


def _fps(xyz_c, npoint):
    b, _, n = xyz_c.shape
    out = pl.pallas_call(
        _fps_body,
        grid=(1,),
        in_specs=[pl.BlockSpec((b, n), lambda i: (0, 0))] * 3,
        out_specs=pl.BlockSpec((npoint, b, 3), lambda i: (0, 0, 0)),
        out_shape=jax.ShapeDtypeStruct((npoint, b, 3), jnp.float32),
        scratch_shapes=[pltpu.VMEM((b, n), jnp.float32)],
    )(xyz_c[:, 0, :], xyz_c[:, 1, :], xyz_c[:, 2, :])
    return jnp.transpose(out, (1, 0, 2))


def _bq_body(nx_ref, xyzc_ref, out_ref, *, radius, nsample):
    nx = nx_ref[0]
    xyzc = xyzc_ref[0]
    sb = nx.shape[0]
    n = xyzc.shape[1]
    src2 = jnp.sum(nx * nx, axis=1, keepdims=True)
    dst2 = jnp.sum(xyzc * xyzc, axis=0, keepdims=True)
    cross = jax.lax.dot_general(nx, xyzc, (((1,), (0,)), ((), ())),
                                preferred_element_type=jnp.float32)
    sq = src2 + dst2 - 2.0 * cross
    iota = jax.lax.broadcasted_iota(jnp.int32, (sb, n), 1)
    keys = jnp.where(sq > radius * radius, n, iota)
    cols = []
    for _ in range(nsample):
        m = jnp.min(keys, axis=1, keepdims=True)
        cols.append(m)
        keys = jnp.where(keys == m, n, keys)
    idx = jnp.concatenate(cols, axis=1)
    idx = jnp.where(idx == n, cols[0], idx)
    out_ref[0] = idx


def _bq(new_xyz, xyz_c, radius, nsample, sb):
    b, s, _ = new_xyz.shape
    n = xyz_c.shape[2]
    grid = (b, s // sb)
    return pl.pallas_call(
        functools.partial(_bq_body, radius=radius, nsample=nsample),
        grid=grid,
        in_specs=[
            pl.BlockSpec((1, sb, 3), lambda i, j: (i, j, 0)),
            pl.BlockSpec((1, 3, n), lambda i, j: (i, 0, 0)),
        ],
        out_specs=pl.BlockSpec((1, sb, nsample), lambda i, j: (i, j, 0)),
        out_shape=jax.ShapeDtypeStruct((b, s, nsample), jnp.int32),
    )(new_xyz, xyz_c)


def _sc_gather(table, idx, nchunks):
    r, d = table.shape
    m = idx.shape[0]
    nw = 32
    bpw = m // nw
    cs = bpw // nchunks
    mesh = plsc.VectorSubcoreMesh(core_axis_name="c", subcore_axis_name="s")

    @functools.partial(
        pl.kernel, mesh=mesh,
        compiler_params=pltpu.CompilerParams(use_tc_tiling_on_sc=False),
        out_type=jax.ShapeDtypeStruct((m, d), jnp.float32),
        scratch_types=[
            pltpu.VMEM((cs,), jnp.int32),
            pltpu.VMEM((cs, d), jnp.float32),
            pltpu.SemaphoreType.DMA,
        ],
    )
    def k(table_hbm, idx_hbm, out_hbm, idx_v, rows_v, sem):
        wid = lax.axis_index("s") * 2 + lax.axis_index("c")
        base = wid * bpw
        for c in range(nchunks):
            off = base + c * cs
            pltpu.sync_copy(idx_hbm.at[pl.ds(off, cs)], idx_v)
            pltpu.async_copy(table_hbm.at[idx_v], rows_v, sem).wait()
            pltpu.sync_copy(rows_v, out_hbm.at[pl.ds(off, cs)])

    return k(table, idx)


def _sa_mlp_body(g_ref, nx_ref, *refs):
    nl = (len(refs) - 1) // 4
    out_ref = refs[-1]
    _, sb, ns, c = g_ref.shape
    x = g_ref[0].reshape(sb * ns, c)
    nx = nx_ref[0]
    for li in range(nl):
        w, bb, g, be = refs[4 * li:4 * li + 4]
        x = jax.lax.dot_general(x, w[...], (((1,), (1,)), ((), ())),
                                preferred_element_type=jnp.float32) + bb[...]
        if li == 0:
            corr = jax.lax.dot_general(
                nx, w[:, :3], (((1,), (1,)), ((), ())),
                preferred_element_type=jnp.float32)
            x = (x.reshape(sb, ns, -1) - corr[:, None, :]).reshape(sb * ns, -1)
        x = x * _BNS * g[...] + be[...]
        x = jnp.maximum(x, 0.0)
    x = x.reshape(sb, ns, x.shape[-1])
    out_ref[0] = jnp.max(x, axis=1)


def _sa_mlp(grouped, new_xyz, layers, sb):
    b, s, ns, c = grouped.shape
    cout = layers[-1]['W'].shape[0]
    grid = (b, s // sb)
    in_specs = [
        pl.BlockSpec((1, sb, ns, c), lambda i, j: (i, j, 0, 0)),
        pl.BlockSpec((1, sb, 3), lambda i, j: (i, j, 0)),
    ]
    args = [grouped, new_xyz]
    for li, p in enumerate(layers):
        co, ci = p['W'].shape
        w = p['W']
        if li == 0 and ci != c:
            w = jnp.pad(w, ((0, 0), (0, c - ci)))
        in_specs += [
            pl.BlockSpec((co, c if li == 0 else ci), lambda i, j: (0, 0)),
            pl.BlockSpec((co,), lambda i, j: (0,)),
            pl.BlockSpec((co,), lambda i, j: (0,)),
            pl.BlockSpec((co,), lambda i, j: (0,)),
        ]
        args += [w, p['b'], p['g'], p['be']]
    return pl.pallas_call(
        _sa_mlp_body,
        grid=grid,
        in_specs=in_specs,
        out_specs=pl.BlockSpec((1, sb, cout), lambda i, j: (i, j, 0)),
        out_shape=jax.ShapeDtypeStruct((b, s, cout), jnp.float32),
    )(*args)


def _fp_body(x1_ref, x2c_ref, p2_ref, *refs, has_p1, nl):
    if has_p1:
        p1_ref = refs[0]
        refs = refs[1:]
    out_ref = refs[-1]
    x1 = x1_ref[0]
    x2c = x2c_ref[0]
    p2 = p2_ref[0]
    nb = x1.shape[0]
    s = x2c.shape[1]
    src2 = jnp.sum(x1 * x1, axis=1, keepdims=True)
    dst2 = jnp.sum(x2c * x2c, axis=0, keepdims=True)
    cross = jax.lax.dot_general(x1, x2c, (((1,), (0,)), ((), ())),
                                preferred_element_type=jnp.float32)
    d = src2 + dst2 - 2.0 * cross
    iota = jax.lax.broadcasted_iota(jnp.int32, (nb, s), 1)
    ws = []
    poss = []
    for _ in range(3):
        m = jnp.min(d, axis=1, keepdims=True)
        pos = jnp.min(jnp.where(d == m, iota, s), axis=1, keepdims=True)
        ws.append(1.0 / (m + 1e-8))
        poss.append(pos)
        d = jnp.where(iota == pos, jnp.float32(3.4e38), d)
    norm = ws[0] + ws[1] + ws[2]
    wmat = jnp.zeros((nb, s), jnp.float32)
    for k in range(3):
        wmat = jnp.where(iota == poss[k], ws[k] / norm, wmat)
    x = jax.lax.dot_general(wmat, p2, (((1,), (0,)), ((), ())),
                            preferred_element_type=jnp.float32)
    if has_p1:
        x = jnp.concatenate([p1_ref[0], x], axis=1)
    for li in range(nl):
        w, bb, g, be = refs[4 * li:4 * li + 4]
        x = jax.lax.dot_general(x, w[...], (((1,), (1,)), ((), ())),
                                preferred_element_type=jnp.float32) + bb[...]
        x = x * _BNS * g[...] + be[...]
        x = jnp.maximum(x, 0.0)
    out_ref[0] = x


def _fp(x1, x2_c, p1, p2, layers, nb):
    b, n, _ = x1.shape
    s = x2_c.shape[2]
    c2 = p2.shape[2]
    cout = layers[-1]['W'].shape[0]
    grid = (b, n // nb)
    in_specs = [
        pl.BlockSpec((1, nb, 3), lambda i, j: (i, j, 0)),
        pl.BlockSpec((1, 3, s), lambda i, j: (i, 0, 0)),
        pl.BlockSpec((1, s, c2), lambda i, j: (i, 0, 0)),
    ]
    args = [x1, x2_c, p2]
    if p1 is not None:
        in_specs.append(pl.BlockSpec((1, nb, p1.shape[2]), lambda i, j: (i, j, 0)))
        args.append(p1)
    for p in layers:
        co, ci = p['W'].shape
        in_specs += [
            pl.BlockSpec((co, ci), lambda i, j: (0, 0)),
            pl.BlockSpec((co,), lambda i, j: (0,)),
            pl.BlockSpec((co,), lambda i, j: (0,)),
            pl.BlockSpec((co,), lambda i, j: (0,)),
        ]
        args += [p['W'], p['b'], p['g'], p['be']]
    return pl.pallas_call(
        functools.partial(_fp_body, has_p1=p1 is not None, nl=len(layers)),
        grid=grid,
        in_specs=in_specs,
        out_specs=pl.BlockSpec((1, nb, cout), lambda i, j: (i, j, 0)),
        out_shape=jax.ShapeDtypeStruct((b, n, cout), jnp.float32),
    )(*args)


def _head_body(x_ref, w1_ref, b1_ref, g1_ref, be1_ref, w2_ref, b2_ref, out_ref):
    x = x_ref[0]
    h = jax.lax.dot_general(x, w1_ref[...], (((1,), (1,)), ((), ())),
                            preferred_element_type=jnp.float32) + b1_ref[...]
    h = h * _BNS * g1_ref[...] + be1_ref[...]
    h = jnp.maximum(h, 0.0)
    logits = jax.lax.dot_general(h, w2_ref[...], (((1,), (1,)), ((), ())),
                                 preferred_element_type=jnp.float32) + b2_ref[...]
    out_ref[0] = jax.nn.log_softmax(logits, axis=-1)


def _head(t, p1, p2):
    b, n, c = t.shape
    blk = 1024
    grid = (b, n // blk)
    return pl.pallas_call(
        _head_body,
        grid=grid,
        in_specs=[
            pl.BlockSpec((1, blk, c), lambda i, j: (i, j, 0)),
            pl.BlockSpec((128, c), lambda i, j: (0, 0)),
            pl.BlockSpec((128,), lambda i, j: (0,)),
            pl.BlockSpec((128,), lambda i, j: (0,)),
            pl.BlockSpec((128,), lambda i, j: (0,)),
            pl.BlockSpec((NUM_CLASSES, c), lambda i, j: (0, 0)),
            pl.BlockSpec((NUM_CLASSES,), lambda i, j: (0,)),
        ],
        out_specs=pl.BlockSpec((1, blk, NUM_CLASSES), lambda i, j: (i, j, 0)),
        out_shape=jax.ShapeDtypeStruct((b, n, NUM_CLASSES), jnp.float32),
    )(t, p1['W'], p1['b'], p1['g'], p1['be'], p2['W'], p2['b'])


def _index_points(points, idx):
    return jax.vmap(lambda p, i: p[i])(points, idx)


def _sa(xyz_c, pts_t, npoint, radius, nsample, layers, sb, nchunks):
    b, _, n = xyz_c.shape
    new_xyz = _fps(xyz_c, npoint)
    idx = _bq(new_xyz, xyz_c, radius, nsample, sb)
    xyz_t = jnp.transpose(xyz_c, (0, 2, 1))
    table = jnp.concatenate([xyz_t, pts_t], axis=-1)
    c = table.shape[-1]
    dp = -(-(c) // 16) * 16
    tablep = jnp.pad(table, ((0, 0), (0, 0), (0, dp - c))).reshape(b * n, dp)
    idxf = (idx + (jnp.arange(b, dtype=jnp.int32) * n)[:, None, None]).reshape(-1)
    g = _sc_gather(tablep, idxf, nchunks).reshape(b, npoint, nsample, dp)
    return new_xyz, _sa_mlp(g, new_xyz, layers, sb)


def kernel(xyz, params):
    l0_xyz_c = xyz[:, :3, :]
    l0_pts_t = jnp.transpose(xyz, (0, 2, 1))

    l1_xyz, l1_pts = _sa(l0_xyz_c, l0_pts_t, 1024, 0.1, 32, params['sa1'], sb=256, nchunks=2)
    l1_xyz_c = jnp.transpose(l1_xyz, (0, 2, 1))
    l2_xyz, l2_pts = _sa(l1_xyz_c, l1_pts, 256, 0.2, 32, params['sa2'], sb=256, nchunks=2)
    l2_xyz_c = jnp.transpose(l2_xyz, (0, 2, 1))
    l3_xyz, l3_pts = _sa(l2_xyz_c, l2_pts, 64, 0.4, 32, params['sa3'], sb=64, nchunks=1)
    l3_xyz_c = jnp.transpose(l3_xyz, (0, 2, 1))
    l4_xyz, l4_pts = _sa(l3_xyz_c, l3_pts, 16, 0.8, 32, params['sa4'], sb=16, nchunks=1)
    l4_xyz_c = jnp.transpose(l4_xyz, (0, 2, 1))

    l3_pts = _fp(l3_xyz, l4_xyz_c, l3_pts, l4_pts, params['fp4'], nb=64)
    l2_pts = _fp(l2_xyz, l3_xyz_c, l2_pts, l3_pts, params['fp3'], nb=256)
    l1_pts = _fp(l1_xyz, l2_xyz_c, l1_pts, l2_pts, params['fp2'], nb=512)
    l0_out = _fp(l0_pts_t[..., :3], l1_xyz_c, None, l1_pts, params['fp1'], nb=512)

    return _head(l0_out, params['head1'], params['head2'])


# speedup vs baseline: 16.4283x; 1.0026x over previous
import functools

import jax
import jax.numpy as jnp
import numpy as np
from jax import lax
from jax.experimental import pallas as pl
from jax.experimental.pallas import tpu as pltpu
from jax.experimental.pallas import tpu_sc as plsc

NUM_CLASSES = 13
BN_EPS = 1e-5
_BNS = float(1.0 / np.sqrt(np.float32(1.0 + BN_EPS), dtype=np.float32))


def _fps_body(x_ref, y_ref, z_ref, out_ref, dist_ref):
    b, n = x_ref.shape
    x = x_ref[...]
    y = y_ref[...]
    z = z_ref[...]
    iota = jax.lax.broadcasted_iota(jnp.int32, (b, n), 1)
    dist_ref[...] = jnp.full((b, n), 1e10, jnp.float32)
    npoint = out_ref.shape[0]

    def step(t, far):
        oh = iota == far
        cx = jnp.sum(jnp.where(oh, x, 0.0), axis=1, keepdims=True)
        cy = jnp.sum(jnp.where(oh, y, 0.0), axis=1, keepdims=True)
        cz = jnp.sum(jnp.where(oh, z, 0.0), axis=1, keepdims=True)
        out_ref[t] = jnp.concatenate([cx, cy, cz], axis=1)
        dx = x - cx
        dy = y - cy
        dz = z - cz
        d = dx * dx + dy * dy + dz * dz
        dist = jnp.minimum(dist_ref[...], d)
        dist_ref[...] = dist
        m = jnp.max(dist, axis=1, keepdims=True)
        return jnp.min(jnp.where(dist == m, iota, n), axis=1, keepdims=True)

    jax.lax.fori_loop(0, npoint, step, jnp.zeros((b, 1), jnp.int32))


def _fps(xyz_c, npoint):
    b, _, n = xyz_c.shape
    out = pl.pallas_call(
        _fps_body,
        grid=(1,),
        in_specs=[pl.BlockSpec((b, n), lambda i: (0, 0))] * 3,
        out_specs=pl.BlockSpec((npoint, b, 3), lambda i: (0, 0, 0)),
        out_shape=jax.ShapeDtypeStruct((npoint, b, 3), jnp.float32),
        scratch_shapes=[pltpu.VMEM((b, n), jnp.float32)],
    )(xyz_c[:, 0, :], xyz_c[:, 1, :], xyz_c[:, 2, :])
    return jnp.transpose(out, (1, 0, 2))


def _bq_body(nx_ref, xyzc_ref, out_ref, *, radius, nsample):
    nx = nx_ref[0]
    xyzc = xyzc_ref[0]
    sb = nx.shape[0]
    n = xyzc.shape[1]
    src2 = jnp.sum(nx * nx, axis=1, keepdims=True)
    dst2 = jnp.sum(xyzc * xyzc, axis=0, keepdims=True)
    cross = jax.lax.dot_general(nx, xyzc, (((1,), (0,)), ((), ())),
                                preferred_element_type=jnp.float32)
    sq = src2 + dst2 - 2.0 * cross
    iota = jax.lax.broadcasted_iota(jnp.int32, (sb, n), 1)
    keys = jnp.where(sq > radius * radius, n, iota)
    cols = []
    for _ in range(nsample):
        m = jnp.min(keys, axis=1, keepdims=True)
        cols.append(m)
        keys = jnp.where(keys == m, n, keys)
    idx = jnp.concatenate(cols, axis=1)
    idx = jnp.where(idx == n, cols[0], idx)
    out_ref[0] = idx


def _bq(new_xyz, xyz_c, radius, nsample, sb):
    b, s, _ = new_xyz.shape
    n = xyz_c.shape[2]
    grid = (b, s // sb)
    return pl.pallas_call(
        functools.partial(_bq_body, radius=radius, nsample=nsample),
        grid=grid,
        in_specs=[
            pl.BlockSpec((1, sb, 3), lambda i, j: (i, j, 0)),
            pl.BlockSpec((1, 3, n), lambda i, j: (i, 0, 0)),
        ],
        out_specs=pl.BlockSpec((1, sb, nsample), lambda i, j: (i, j, 0)),
        out_shape=jax.ShapeDtypeStruct((b, s, nsample), jnp.int32),
    )(new_xyz, xyz_c)


def _sc_gather(table, idx, nchunks):
    r, d = table.shape
    m = idx.shape[0]
    nw = 32
    bpw = m // nw
    cs = bpw // nchunks
    mesh = plsc.VectorSubcoreMesh(core_axis_name="c", subcore_axis_name="s")

    @functools.partial(
        pl.kernel, mesh=mesh,
        compiler_params=pltpu.CompilerParams(use_tc_tiling_on_sc=False),
        out_type=jax.ShapeDtypeStruct((m, d), jnp.float32),
        scratch_types=[
            pltpu.VMEM((cs,), jnp.int32),
            pltpu.VMEM((cs, d), jnp.float32),
            pltpu.SemaphoreType.DMA,
        ],
    )
    def k(table_hbm, idx_hbm, out_hbm, idx_v, rows_v, sem):
        wid = lax.axis_index("s") * 2 + lax.axis_index("c")
        base = wid * bpw
        for c in range(nchunks):
            off = base + c * cs
            pltpu.sync_copy(idx_hbm.at[pl.ds(off, cs)], idx_v)
            pltpu.async_copy(table_hbm.at[idx_v], rows_v, sem).wait()
            pltpu.sync_copy(rows_v, out_hbm.at[pl.ds(off, cs)])

    return k(table, idx)


def _sa_mlp_body(g_ref, nx_ref, *refs):
    nl = (len(refs) - 1) // 4
    out_ref = refs[-1]
    _, sb, ns, c = g_ref.shape
    x = g_ref[0].reshape(sb * ns, c)
    nx = nx_ref[0]
    for li in range(nl):
        w, bb, g, be = refs[4 * li:4 * li + 4]
        x = jax.lax.dot_general(x, w[...], (((1,), (1,)), ((), ())),
                                preferred_element_type=jnp.float32) + bb[...]
        if li == 0:
            corr = jax.lax.dot_general(
                nx, w[:, :3], (((1,), (1,)), ((), ())),
                preferred_element_type=jnp.float32)
            x = (x.reshape(sb, ns, -1) - corr[:, None, :]).reshape(sb * ns, -1)
        x = x * _BNS * g[...] + be[...]
        x = jnp.maximum(x, 0.0)
    x = x.reshape(sb, ns, x.shape[-1])
    out_ref[0] = jnp.max(x, axis=1)


def _sa_mlp(grouped, new_xyz, layers, sb):
    b, s, ns, c = grouped.shape
    cout = layers[-1]['W'].shape[0]
    grid = (b, s // sb)
    in_specs = [
        pl.BlockSpec((1, sb, ns, c), lambda i, j: (i, j, 0, 0)),
        pl.BlockSpec((1, sb, 3), lambda i, j: (i, j, 0)),
    ]
    args = [grouped, new_xyz]
    for li, p in enumerate(layers):
        co, ci = p['W'].shape
        w = p['W']
        if li == 0 and ci != c:
            w = jnp.pad(w, ((0, 0), (0, c - ci)))
        in_specs += [
            pl.BlockSpec((co, c if li == 0 else ci), lambda i, j: (0, 0)),
            pl.BlockSpec((co,), lambda i, j: (0,)),
            pl.BlockSpec((co,), lambda i, j: (0,)),
            pl.BlockSpec((co,), lambda i, j: (0,)),
        ]
        args += [w, p['b'], p['g'], p['be']]
    return pl.pallas_call(
        _sa_mlp_body,
        grid=grid,
        in_specs=in_specs,
        out_specs=pl.BlockSpec((1, sb, cout), lambda i, j: (i, j, 0)),
        out_shape=jax.ShapeDtypeStruct((b, s, cout), jnp.float32),
    )(*args)


def _fp_body(x1_ref, x2c_ref, p2_ref, *refs, has_p1, nl, head):
    if has_p1:
        p1_ref = refs[0]
        refs = refs[1:]
    out_ref = refs[-1]
    x1 = x1_ref[0]
    x2c = x2c_ref[0]
    p2 = p2_ref[0]
    nb = x1.shape[0]
    s = x2c.shape[1]
    src2 = jnp.sum(x1 * x1, axis=1, keepdims=True)
    dst2 = jnp.sum(x2c * x2c, axis=0, keepdims=True)
    cross = jax.lax.dot_general(x1, x2c, (((1,), (0,)), ((), ())),
                                preferred_element_type=jnp.float32)
    d = src2 + dst2 - 2.0 * cross
    iota = jax.lax.broadcasted_iota(jnp.int32, (nb, s), 1)
    ws = []
    poss = []
    for _ in range(3):
        m = jnp.min(d, axis=1, keepdims=True)
        pos = jnp.min(jnp.where(d == m, iota, s), axis=1, keepdims=True)
        ws.append(1.0 / (m + 1e-8))
        poss.append(pos)
        d = jnp.where(iota == pos, jnp.float32(3.4e38), d)
    norm = ws[0] + ws[1] + ws[2]
    wmat = jnp.zeros((nb, s), jnp.float32)
    for k in range(3):
        wmat = jnp.where(iota == poss[k], ws[k] / norm, wmat)
    x = jax.lax.dot_general(wmat, p2, (((1,), (0,)), ((), ())),
                            preferred_element_type=jnp.float32)
    if has_p1:
        x = jnp.concatenate([p1_ref[0], x], axis=1)
    for li in range(nl):
        w, bb, g, be = refs[4 * li:4 * li + 4]
        x = jax.lax.dot_general(x, w[...], (((1,), (1,)), ((), ())),
                                preferred_element_type=jnp.float32) + bb[...]
        x = x * _BNS * g[...] + be[...]
        x = jnp.maximum(x, 0.0)
    if head:
        w1, b1, g1, be1, w2, b2 = refs[4 * nl:4 * nl + 6]
        h = jax.lax.dot_general(x, w1[...], (((1,), (1,)), ((), ())),
                                preferred_element_type=jnp.float32) + b1[...]
        h = h * _BNS * g1[...] + be1[...]
        h = jnp.maximum(h, 0.0)
        logits = jax.lax.dot_general(h, w2[...], (((1,), (1,)), ((), ())),
                                     preferred_element_type=jnp.float32) + b2[...]
        x = jax.nn.log_softmax(logits, axis=-1)
    out_ref[0] = x


def _fp(x1, x2_c, p1, p2, layers, nb, head=None):
    b, n, _ = x1.shape
    s = x2_c.shape[2]
    c2 = p2.shape[2]
    cout = NUM_CLASSES if head is not None else layers[-1]['W'].shape[0]
    grid = (b, n // nb)
    in_specs = [
        pl.BlockSpec((1, nb, 3), lambda i, j: (i, j, 0)),
        pl.BlockSpec((1, 3, s), lambda i, j: (i, 0, 0)),
        pl.BlockSpec((1, s, c2), lambda i, j: (i, 0, 0)),
    ]
    args = [x1, x2_c, p2]
    if p1 is not None:
        in_specs.append(pl.BlockSpec((1, nb, p1.shape[2]), lambda i, j: (i, j, 0)))
        args.append(p1)
    for p in layers:
        co, ci = p['W'].shape
        in_specs += [
            pl.BlockSpec((co, ci), lambda i, j: (0, 0)),
            pl.BlockSpec((co,), lambda i, j: (0,)),
            pl.BlockSpec((co,), lambda i, j: (0,)),
            pl.BlockSpec((co,), lambda i, j: (0,)),
        ]
        args += [p['W'], p['b'], p['g'], p['be']]
    if head is not None:
        p1h, p2h = head
        in_specs += [
            pl.BlockSpec((128, 128), lambda i, j: (0, 0)),
            pl.BlockSpec((128,), lambda i, j: (0,)),
            pl.BlockSpec((128,), lambda i, j: (0,)),
            pl.BlockSpec((128,), lambda i, j: (0,)),
            pl.BlockSpec((NUM_CLASSES, 128), lambda i, j: (0, 0)),
            pl.BlockSpec((NUM_CLASSES,), lambda i, j: (0,)),
        ]
        args += [p1h['W'], p1h['b'], p1h['g'], p1h['be'], p2h['W'], p2h['b']]
    return pl.pallas_call(
        functools.partial(_fp_body, has_p1=p1 is not None, nl=len(layers),
                          head=head is not None),
        grid=grid,
        in_specs=in_specs,
        out_specs=pl.BlockSpec((1, nb, cout), lambda i, j: (i, j, 0)),
        out_shape=jax.ShapeDtypeStruct((b, n, cout), jnp.float32),
    )(*args)


def _head_body(x_ref, w1_ref, b1_ref, g1_ref, be1_ref, w2_ref, b2_ref, out_ref):
    x = x_ref[0]
    h = jax.lax.dot_general(x, w1_ref[...], (((1,), (1,)), ((), ())),
                            preferred_element_type=jnp.float32) + b1_ref[...]
    h = h * _BNS * g1_ref[...] + be1_ref[...]
    h = jnp.maximum(h, 0.0)
    logits = jax.lax.dot_general(h, w2_ref[...], (((1,), (1,)), ((), ())),
                                 preferred_element_type=jnp.float32) + b2_ref[...]
    out_ref[0] = jax.nn.log_softmax(logits, axis=-1)


def _head(t, p1, p2):
    b, n, c = t.shape
    blk = 1024
    grid = (b, n // blk)
    return pl.pallas_call(
        _head_body,
        grid=grid,
        in_specs=[
            pl.BlockSpec((1, blk, c), lambda i, j: (i, j, 0)),
            pl.BlockSpec((128, c), lambda i, j: (0, 0)),
            pl.BlockSpec((128,), lambda i, j: (0,)),
            pl.BlockSpec((128,), lambda i, j: (0,)),
            pl.BlockSpec((128,), lambda i, j: (0,)),
            pl.BlockSpec((NUM_CLASSES, c), lambda i, j: (0, 0)),
            pl.BlockSpec((NUM_CLASSES,), lambda i, j: (0,)),
        ],
        out_specs=pl.BlockSpec((1, blk, NUM_CLASSES), lambda i, j: (i, j, 0)),
        out_shape=jax.ShapeDtypeStruct((b, n, NUM_CLASSES), jnp.float32),
    )(t, p1['W'], p1['b'], p1['g'], p1['be'], p2['W'], p2['b'])


def _index_points(points, idx):
    return jax.vmap(lambda p, i: p[i])(points, idx)


def _sa(xyz_c, pts_t, npoint, radius, nsample, layers, sb, nchunks):
    b, _, n = xyz_c.shape
    new_xyz = _fps(xyz_c, npoint)
    idx = _bq(new_xyz, xyz_c, radius, nsample, sb)
    xyz_t = jnp.transpose(xyz_c, (0, 2, 1))
    table = jnp.concatenate([xyz_t, pts_t], axis=-1)
    c = table.shape[-1]
    dp = -(-(c) // 16) * 16
    tablep = jnp.pad(table, ((0, 0), (0, 0), (0, dp - c))).reshape(b * n, dp)
    idxf = (idx + (jnp.arange(b, dtype=jnp.int32) * n)[:, None, None]).reshape(-1)
    g = _sc_gather(tablep, idxf, nchunks).reshape(b, npoint, nsample, dp)
    return new_xyz, _sa_mlp(g, new_xyz, layers, sb)


def kernel(xyz, params):
    l0_xyz_c = xyz[:, :3, :]
    l0_pts_t = jnp.transpose(xyz, (0, 2, 1))

    l1_xyz, l1_pts = _sa(l0_xyz_c, l0_pts_t, 1024, 0.1, 32, params['sa1'], sb=256, nchunks=2)
    l1_xyz_c = jnp.transpose(l1_xyz, (0, 2, 1))
    l2_xyz, l2_pts = _sa(l1_xyz_c, l1_pts, 256, 0.2, 32, params['sa2'], sb=256, nchunks=2)
    l2_xyz_c = jnp.transpose(l2_xyz, (0, 2, 1))
    l3_xyz, l3_pts = _sa(l2_xyz_c, l2_pts, 64, 0.4, 32, params['sa3'], sb=64, nchunks=1)
    l3_xyz_c = jnp.transpose(l3_xyz, (0, 2, 1))
    l4_xyz, l4_pts = _sa(l3_xyz_c, l3_pts, 16, 0.8, 32, params['sa4'], sb=16, nchunks=1)
    l4_xyz_c = jnp.transpose(l4_xyz, (0, 2, 1))

    l3_pts = _fp(l3_xyz, l4_xyz_c, l3_pts, l4_pts, params['fp4'], nb=64)
    l2_pts = _fp(l2_xyz, l3_xyz_c, l2_pts, l3_pts, params['fp3'], nb=256)
    l1_pts = _fp(l1_xyz, l2_xyz_c, l1_pts, l2_pts, params['fp2'], nb=512)
    return _fp(l0_pts_t[..., :3], l1_xyz_c, None, l1_pts, params['fp1'], nb=512,
               head=(params['head1'], params['head2']))
